# Initial kernel scaffold; baseline (speedup 1.0000x reference)
import jax
import jax.numpy as jnp
from jax.experimental import pallas as pl


def kernel(x, edge_index, edge_weight, W, b, gamma, beta):
    raise NotImplementedError("write your pallas kernel here")


# speedup vs baseline: 5.3601x; 5.3601x over previous
import functools

import jax
import jax.numpy as jnp
from jax import lax
from jax.experimental import pallas as pl
from jax.experimental.pallas import tpu as pltpu
from jax.experimental.pallas import tpu_sc as plsc

N = 10000
E = 320000
D = 128
EPS = 1e-5

CHUNK = 128
NUM_CHUNKS = E // CHUNK
NC = 2
NS = 16
NW = NC * NS
MAX_K = -(-NUM_CHUNKS // NW)
NPAD = 10240
ROWS_PER_TILE = NPAD // NS


def _sc_spmm_body(x_hbm, col_hbm, row_hbm, w_hbm, out_hbm,
                  col_v, row_v, w_v, rows_v, acc, sem):
    cid = lax.axis_index("c")
    sid = lax.axis_index("s")
    wid = sid * NC + cid

    zero16 = jnp.zeros((16,), jnp.float32)

    def zrow(r, carry):
        for j in range(D // 16):
            rows_v[r, pl.ds(16 * j, 16)] = zero16
        return carry

    lax.fori_loop(0, CHUNK, zrow, 0)

    zbase = sid * ROWS_PER_TILE
    full = ROWS_PER_TILE // CHUNK
    for k in range(full):
        pltpu.sync_copy(rows_v, acc.at[pl.ds(zbase + CHUNK * k, CHUNK)])

    plsc.subcore_barrier()

    def chunk_body(c):
        base = c * CHUNK
        pltpu.sync_copy(col_hbm.at[pl.ds(base, CHUNK)], col_v)
        pltpu.sync_copy(row_hbm.at[pl.ds(base, CHUNK)], row_v)
        pltpu.sync_copy(w_hbm.at[pl.ds(base, CHUNK)], w_v)
        pltpu.sync_copy(x_hbm.at[col_v], rows_v)

        def sgroup(g, carry):
            wv = w_v[pl.ds(16 * g, 16)]
            for rp in range(16):
                wr = wv[rp]
                r = 16 * g + rp
                for j in range(D // 16):
                    sl = pl.ds(16 * j, 16)
                    rows_v[r, sl] = rows_v[r, sl] * wr
            return carry

        lax.fori_loop(0, CHUNK // 16, sgroup, 0)
        pltpu.sync_copy(rows_v, acc.at[row_v], add=True)

    def k_body(k, carry):
        c = wid + NW * k

        @pl.when(c < NUM_CHUNKS)
        def _():
            chunk_body(c)

        return carry

    lax.fori_loop(0, MAX_K, k_body, 0)

    plsc.subcore_barrier()

    for k in range(full):
        r0 = zbase + CHUNK * k
        pltpu.sync_copy(acc.at[pl.ds(r0, CHUNK)], rows_v)
        pltpu.sync_copy(rows_v, out_hbm.at[cid, pl.ds(r0, CHUNK)])


_sc_spmm = functools.partial(
    pl.kernel,
    out_type=jax.ShapeDtypeStruct((NC, NPAD, D), jnp.float32),
    mesh=plsc.VectorSubcoreMesh(core_axis_name="c", subcore_axis_name="s"),
    scratch_types=[
        pltpu.VMEM((CHUNK,), jnp.int32),
        pltpu.VMEM((CHUNK,), jnp.int32),
        pltpu.VMEM((CHUNK,), jnp.float32),
        pltpu.VMEM((CHUNK, D), jnp.float32),
        pltpu.VMEM_SHARED((NPAD, D), jnp.float32),
        pltpu.SemaphoreType.DMA,
    ],
)(_sc_spmm_body)


BLK = 1000
NBLK = N // BLK


def _tc_linear_body(agg_ref, wt_ref, b_ref, h_ref, stats_ref):
    i = pl.program_id(0)
    a = agg_ref[0] + agg_ref[1]
    h = jnp.dot(a, wt_ref[...], preferred_element_type=jnp.float32) + b_ref[...]
    h_ref[...] = h

    @pl.when(i == 0)
    def _():
        stats_ref[...] = jnp.zeros_like(stats_ref)

    stats_ref[0:1, :] += jnp.sum(h, axis=0, keepdims=True)
    stats_ref[1:2, :] += jnp.sum(h * h, axis=0, keepdims=True)


def _tc_linear(agg2, wt, b2):
    return pl.pallas_call(
        _tc_linear_body,
        grid=(NBLK,),
        in_specs=[
            pl.BlockSpec((NC, BLK, D), lambda i: (0, i, 0)),
            pl.BlockSpec((D, D), lambda i: (0, 0)),
            pl.BlockSpec((1, D), lambda i: (0, 0)),
        ],
        out_specs=[
            pl.BlockSpec((BLK, D), lambda i: (i, 0)),
            pl.BlockSpec((8, D), lambda i: (0, 0)),
        ],
        out_shape=[
            jax.ShapeDtypeStruct((N, D), jnp.float32),
            jax.ShapeDtypeStruct((8, D), jnp.float32),
        ],
    )(agg2, wt, b2)


def _tc_bn_body(h_ref, stats_ref, gamma_ref, beta_ref, out_ref):
    mean = stats_ref[0:1, :] / N
    var = stats_ref[1:2, :] / N - mean * mean
    inv = lax.rsqrt(var + EPS)
    scale = inv * gamma_ref[...]
    shift = beta_ref[...] - mean * scale
    out_ref[...] = h_ref[...] * scale + shift


def _tc_bn(h, stats, gamma2, beta2):
    return pl.pallas_call(
        _tc_bn_body,
        grid=(NBLK,),
        in_specs=[
            pl.BlockSpec((BLK, D), lambda i: (i, 0)),
            pl.BlockSpec((8, D), lambda i: (0, 0)),
            pl.BlockSpec((1, D), lambda i: (0, 0)),
            pl.BlockSpec((1, D), lambda i: (0, 0)),
        ],
        out_specs=pl.BlockSpec((BLK, D), lambda i: (i, 0)),
        out_shape=jax.ShapeDtypeStruct((N, D), jnp.float32),
    )(h, stats, gamma2, beta2)


@jax.jit
def kernel(x, edge_index, edge_weight, W, b, gamma, beta):
    row = edge_index[0].astype(jnp.int32)
    col = edge_index[1].astype(jnp.int32)
    agg2 = _sc_spmm(x, col, row, edge_weight)
    h, stats = _tc_linear(agg2, W.T, b.reshape(1, D))
    return _tc_bn(h, stats, gamma.reshape(1, D), beta.reshape(1, D))
